# baseline (speedup 1.0000x reference)
import functools

import jax
import jax.numpy as jnp
from jax import lax
from jax.experimental import pallas as pl
from jax.experimental.pallas import tpu as pltpu
from jax.experimental.pallas import tpu_sc as plsc

N, T = 4096, 200
D = 64
V = 1000000
B = N * T
NC, NS = 2, 16
NW = NC * NS
K = 128
NBLK = B // K
BLK_PER_W = NBLK // NW
NBUF = 4
NGRP = BLK_PER_W // NBUF
NB_N = N // K


NBLK1 = 7813


@functools.partial(
    pl.kernel,
    mesh=plsc.VectorSubcoreMesh(core_axis_name="c", subcore_axis_name="s"),
    out_type=jax.ShapeDtypeStruct((V, 2 * D), jnp.float32),
    compiler_params=pltpu.CompilerParams(
        use_tc_tiling_on_sc=True, needs_layout_passes=False),
    scratch_types=(
        [pltpu.VMEM((D, K), jnp.float32)] * 4
        + [pltpu.VMEM((K, 2 * D), jnp.float32)] * 4
        + [pltpu.SemaphoreType.DMA] * 8
    ),
)
def _transpose_kernel(tT_hbm, tail_hbm, w_hbm, *scratch):
    ibuf = scratch[:4]
    obuf = scratch[4:8]
    isem = scratch[8:12]
    osem = scratch[12:16]
    wid = lax.axis_index("s") * NC + lax.axis_index("c")
    nfull = 244 + (wid < 4).astype(jnp.int32)
    c0 = 244 * wid + lax.min(wid, 4)

    lane1 = lax.iota(jnp.int32, 16)
    dsel = [lane1 + 16 * j for j in range(D // 16)]

    def start_in(c, b):
        pltpu.async_copy(tT_hbm.at[:, pl.ds(c * K, K)], ibuf[b], isem[b])

    def wait_in(c, b):
        pltpu.make_async_copy(tT_hbm.at[:, pl.ds(c * K, K)], ibuf[b], isem[b]).wait()

    def start_out(c, b):
        pltpu.async_copy(obuf[b], w_hbm.at[pl.ds(c * K, K)], osem[b])

    def wait_out(c, b):
        pltpu.make_async_copy(obuf[b], w_hbm.at[pl.ds(c * K, K)], osem[b]).wait()

    def transpose_blk(b):
        @plsc.parallel_loop(0, K, unroll=4)
        def _per_l(l):
            col = lax.broadcast(l, (16,))
            for j in range(D // 16):
                vals = plsc.load_gather(ibuf[b], [dsel[j], col])
                obuf[b][l, pl.ds(16 * j, 16)] = vals

    for b in range(4):
        start_in(c0 + b, b)

    def body(i, carry):
        for b in range(4):
            c = c0 + 4 * i + b
            @pl.when(c < c0 + nfull)
            def _():
                wait_in(c, b)
                @pl.when(c >= c0 + 4)
                def _():
                    wait_out(c - 4, b)
                transpose_blk(b)
                start_out(c, b)
                @pl.when(c + 4 < c0 + nfull)
                def _():
                    start_in(c + 4, b)
        return carry

    lax.fori_loop(0, 62, body, 0)
    for b in range(4):
        @pl.when(nfull - 4 + b >= 0)
        def _():
            wait_out(c0 + nfull - 4 + b, b)

    @pl.when(wid == NW - 1)
    def _():
        pltpu.sync_copy(tail_hbm, obuf[0].at[pl.ds(0, D)])
        pltpu.sync_copy(obuf[0].at[pl.ds(0, D)], w_hbm.at[pl.ds((NBLK1 - 1) * K, D)])


@functools.partial(
    pl.kernel,
    mesh=plsc.VectorSubcoreMesh(core_axis_name="c", subcore_axis_name="s"),
    out_type=jax.ShapeDtypeStruct((T, D, N), jnp.float32),
    compiler_params=pltpu.CompilerParams(
        use_tc_tiling_on_sc=True, needs_layout_passes=False),
    scratch_types=(
        [pltpu.VMEM((BLK_PER_W, K), jnp.int32)]
        + [pltpu.VMEM((K, 128), jnp.float32)] * NBUF
        + [pltpu.VMEM((D, K), jnp.float32)] * NBUF
        + [pltpu.SemaphoreType.DMA] * (2 * NBUF)
    ),
)
def _gather_kernel(table_hbm, idx_hbm, out_hbm, idx_v, *scratch):
    rows = scratch[:NBUF]
    tbuf = scratch[NBUF:2 * NBUF]
    gsem = scratch[2 * NBUF:3 * NBUF]
    wsem = scratch[3 * NBUF:]
    wid = lax.axis_index("s") * NC + lax.axis_index("c")
    pltpu.sync_copy(idx_hbm.at[pl.ds(wid * BLK_PER_W, BLK_PER_W)], idx_v)
    base = wid * BLK_PER_W

    lane = lax.iota(jnp.int32, 16)
    row_sel = [lane + 16 * k for k in range(K // 16)]

    def start_gather(g, b):
        pltpu.async_copy(table_hbm.at[idx_v.at[g]], rows[b], gsem[b])

    def wait_gather(g, b):
        pltpu.make_async_copy(table_hbm.at[idx_v.at[g]], rows[b], gsem[b]).wait()

    def _dst(g):
        r = base + g
        return out_hbm.at[r >> 5, :, pl.ds((r & 31) * K, K)]

    def start_wb(g, b):
        pltpu.async_copy(tbuf[b], _dst(g), wsem[b])

    def wait_wb(g, b):
        pltpu.make_async_copy(tbuf[b], _dst(g), wsem[b]).wait()

    def transpose(b):
        @plsc.parallel_loop(0, D, unroll=8)
        def _per_d(d):
            col = lax.broadcast(d, (16,))
            for k in range(K // 16):
                vals = plsc.load_gather(rows[b], [row_sel[k], col])
                tbuf[b][d, pl.ds(16 * k, 16)] = vals

    for b in range(NBUF):
        start_gather(b, b)

    for b in range(NBUF):
        wait_gather(b, b)
        transpose(b)
        start_wb(b, b)
        start_gather(NBUF + b, b)

    def group(i, carry):
        g0 = i * NBUF
        for b in range(NBUF):
            g = g0 + b
            wait_gather(g, b)
            wait_wb(g - NBUF, b)
            transpose(b)
            start_wb(g, b)
            start_gather(g + NBUF, b)
        return carry

    lax.fori_loop(1, NGRP - 1, group, 0)

    g0 = (NGRP - 1) * NBUF
    for b in range(NBUF):
        g = g0 + b
        wait_gather(g, b)
        wait_wb(g - NBUF, b)
        transpose(b)
        start_wb(g, b)
    for b in range(NBUF):
        wait_wb(g0 + b, b)


def kernel(x, W_embed):
    idx = jnp.transpose(x).reshape(NBLK, K).astype(jnp.int32)
    tail = jnp.pad(W_embed[V - D:], ((0, 0), (0, 128 - D)))
    Wp = _transpose_kernel(jnp.transpose(W_embed), tail)
    out = _gather_kernel(Wp, idx)
    return out.transpose(2, 0, 1)


# speedup vs baseline: 1.2235x; 1.2235x over previous
import functools

import jax
import jax.numpy as jnp
from jax import lax
from jax.experimental import pallas as pl
from jax.experimental.pallas import tpu as pltpu
from jax.experimental.pallas import tpu_sc as plsc

N, T = 4096, 200
D = 64
V = 1000000
B = N * T
NC, NS = 2, 16
NW = NC * NS
K = 128
NBLK = B // K
BLK_PER_W = NBLK // NW
NBUF = 4
NGRP = BLK_PER_W // NBUF
NB_N = N // K


@functools.partial(
    pl.kernel,
    mesh=plsc.VectorSubcoreMesh(core_axis_name="c", subcore_axis_name="s"),
    out_type=jax.ShapeDtypeStruct((T, D, N), jnp.float32),
    compiler_params=pltpu.CompilerParams(
        use_tc_tiling_on_sc=True, needs_layout_passes=False),
    scratch_types=(
        [pltpu.VMEM((BLK_PER_W, K), jnp.int32)]
        + [pltpu.VMEM((K, 128), jnp.float32)] * NBUF
        + [pltpu.VMEM((D, K), jnp.float32)] * NBUF
        + [pltpu.SemaphoreType.DMA] * (2 * NBUF)
    ),
)
def _gather_kernel(table_hbm, idx_hbm, out_hbm, idx_v, *scratch):
    rows = scratch[:NBUF]
    tbuf = scratch[NBUF:2 * NBUF]
    gsem = scratch[2 * NBUF:3 * NBUF]
    wsem = scratch[3 * NBUF:]
    wid = lax.axis_index("s") * NC + lax.axis_index("c")
    pltpu.sync_copy(idx_hbm.at[pl.ds(wid * BLK_PER_W, BLK_PER_W)], idx_v)
    base = wid * BLK_PER_W

    lane = lax.iota(jnp.int32, 16)
    row_sel = [lane + 16 * k for k in range(K // 16)]

    def start_gather(g, b):
        pltpu.async_copy(table_hbm.at[idx_v.at[g]], rows[b], gsem[b])

    def wait_gather(g, b):
        pltpu.make_async_copy(table_hbm.at[idx_v.at[g]], rows[b], gsem[b]).wait()

    def _dst(g):
        r = base + g
        return out_hbm.at[r >> 5, :, pl.ds((r & 31) * K, K)]

    def start_wb(g, b):
        pltpu.async_copy(tbuf[b], _dst(g), wsem[b])

    def wait_wb(g, b):
        pltpu.make_async_copy(tbuf[b], _dst(g), wsem[b]).wait()

    def transpose(b):
        @plsc.parallel_loop(0, D, unroll=8)
        def _per_d(d):
            col = lax.broadcast(d, (16,))
            for k in range(K // 16):
                vals = plsc.load_gather(rows[b], [row_sel[k], col])
                tbuf[b][d, pl.ds(16 * k, 16)] = vals

    for b in range(NBUF):
        start_gather(b, b)

    for b in range(NBUF):
        wait_gather(b, b)
        transpose(b)
        start_wb(b, b)
        start_gather(NBUF + b, b)

    def group(i, carry):
        g0 = i * NBUF
        for b in range(NBUF):
            g = g0 + b
            wait_gather(g, b)
            wait_wb(g - NBUF, b)
            transpose(b)
            start_wb(g, b)
            start_gather(g + NBUF, b)
        return carry

    lax.fori_loop(1, NGRP - 1, group, 0)

    g0 = (NGRP - 1) * NBUF
    for b in range(NBUF):
        g = g0 + b
        wait_gather(g, b)
        wait_wb(g - NBUF, b)
        transpose(b)
        start_wb(g, b)
    for b in range(NBUF):
        wait_wb(g0 + b, b)


def kernel(x, W_embed):
    idx = jnp.transpose(x).reshape(NBLK, K).astype(jnp.int32)
    Wp = jnp.pad(W_embed, ((0, 0), (0, 128 - D)))
    out = _gather_kernel(Wp, idx)
    return out.transpose(2, 0, 1)
